# baseline (speedup 1.0000x reference)
import jax
import jax.numpy as jnp
from jax import lax
from jax.experimental import pallas as pl
from jax.experimental.pallas import tpu as pltpu

BATCH = 1024
EMBED_DIM = 32
IMG_SHAPE = (3, 775, 120)
OUT_DIM = 3 * 775 * 120

_UNROLL = 16
_N_T = 4096
_GRID = (OUT_DIM + _N_T - 1) // _N_T


def _body(labels_ref, oh_ref, wT_ref, b_ref, tableT_ref, o_ref, x_sc, panels, sem):
    n = pl.program_id(0)

    @pl.when(n == 0)
    def _issue_gather():
        def _issue(j, c):
            for u in range(_UNROLL):
                i = j * _UNROLL + u
                base = pl.multiple_of((labels_ref[i] // 128) * 128, 128)
                pltpu.make_async_copy(
                    tableT_ref.at[:, pl.ds(base, 128)],
                    panels.at[i],
                    sem.at[0],
                ).start()
            return c

        lax.fori_loop(0, BATCH // _UNROLL, _issue, 0)
        x_sc[:, EMBED_DIM:] = jnp.ones((BATCH, 1), jnp.float32)

    @pl.when(n == 1)
    def _extract():
        pltpu.make_async_copy(panels, panels, sem.at[0]).wait()
        x_sc[:, :EMBED_DIM] = jnp.sum(panels[...] * oh_ref[...], axis=2)

    waug = jnp.concatenate([wT_ref[...], b_ref[...][None, :]], axis=0)
    o_ref[...] = lax.dot_general(
        waug, x_sc[...], (((0,), (1,)), ((), ())),
        preferred_element_type=jnp.float32,
    )


@jax.jit
def kernel(labels, embed_table, W, b):
    tableT = embed_table.T
    WT = W.T
    oh = jax.nn.one_hot(labels % 128, 128, dtype=jnp.float32)
    yT = pl.pallas_call(
        _body,
        grid=(_GRID + 1,),
        in_specs=[
            pl.BlockSpec(memory_space=pltpu.SMEM),
            pl.BlockSpec((BATCH, 1, 128), lambda n: (0, 0, 0)),
            pl.BlockSpec((EMBED_DIM, _N_T), lambda n: (0, jnp.maximum(n - 1, 0))),
            pl.BlockSpec((_N_T,), lambda n: (jnp.maximum(n - 1, 0),)),
            pl.BlockSpec(memory_space=pl.ANY),
        ],
        out_specs=pl.BlockSpec((_N_T, BATCH), lambda n: (jnp.maximum(n - 1, 0), 0)),
        out_shape=jax.ShapeDtypeStruct((OUT_DIM, BATCH), jnp.float32),
        scratch_shapes=[
            pltpu.VMEM((BATCH, EMBED_DIM + 1), jnp.float32),
            pltpu.VMEM((BATCH, EMBED_DIM, 128), jnp.float32),
            pltpu.SemaphoreType.DMA((1,)),
        ],
        compiler_params=pltpu.CompilerParams(
            dimension_semantics=("arbitrary",),
        ),
    )(labels, oh.reshape(BATCH, 1, 128), WT, b, tableT)
    y = yT.reshape(*IMG_SHAPE, BATCH).transpose(3, 0, 1, 2)
    return y


# speedup vs baseline: 1.0125x; 1.0125x over previous
import jax
import jax.numpy as jnp
from jax import lax
from jax.experimental import pallas as pl
from jax.experimental.pallas import tpu as pltpu

BATCH = 1024
EMBED_DIM = 32
IMG_SHAPE = (3, 775, 120)
OUT_DIM = 3 * 775 * 120

_UNROLL = 16
_N_T = 4096
_GRID = (OUT_DIM + _N_T - 1) // _N_T


def _body(labels_ref, oh_ref, wT_ref, b_ref, tableT_ref, o_ref, x_sc, panels, sem):
    n = pl.program_id(0)

    @pl.when(n == 0)
    def _issue_gather():
        def _issue(j, c):
            for u in range(_UNROLL):
                i = j * _UNROLL + u
                base = pl.multiple_of((labels_ref[i] // 128) * 128, 128)
                pltpu.make_async_copy(
                    tableT_ref.at[:, pl.ds(base, 128)],
                    panels.at[i],
                    sem.at[0],
                ).start()
            return c

        lax.fori_loop(0, BATCH // _UNROLL, _issue, 0)
        x_sc[:, EMBED_DIM:] = jnp.ones((BATCH, 1), jnp.float32)
        pltpu.make_async_copy(panels, panels, sem.at[0]).wait()
        x_sc[:, :EMBED_DIM] = jnp.sum(panels[...] * oh_ref[...], axis=2)

    waug = jnp.concatenate([wT_ref[...], b_ref[...][None, :]], axis=0)
    o_ref[...] = lax.dot_general(
        waug, x_sc[...], (((0,), (1,)), ((), ())),
        preferred_element_type=jnp.float32,
    )


@jax.jit
def kernel(labels, embed_table, W, b):
    tableT = embed_table.T
    WT = W.T
    oh = jax.nn.one_hot(labels % 128, 128, dtype=jnp.float32)
    yT = pl.pallas_call(
        _body,
        grid=(_GRID,),
        in_specs=[
            pl.BlockSpec(memory_space=pltpu.SMEM),
            pl.BlockSpec((BATCH, 1, 128), lambda n: (0, 0, 0)),
            pl.BlockSpec((EMBED_DIM, _N_T), lambda n: (0, n)),
            pl.BlockSpec((_N_T,), lambda n: (n,)),
            pl.BlockSpec(memory_space=pl.ANY),
        ],
        out_specs=pl.BlockSpec((_N_T, BATCH), lambda n: (n, 0)),
        out_shape=jax.ShapeDtypeStruct((OUT_DIM, BATCH), jnp.float32),
        scratch_shapes=[
            pltpu.VMEM((BATCH, EMBED_DIM + 1), jnp.float32),
            pltpu.VMEM((BATCH, EMBED_DIM, 128), jnp.float32),
            pltpu.SemaphoreType.DMA((1,)),
        ],
        compiler_params=pltpu.CompilerParams(
            dimension_semantics=("arbitrary",),
        ),
    )(labels, oh.reshape(BATCH, 1, 128), WT, b, tableT)
    y = yT.reshape(*IMG_SHAPE, BATCH).transpose(3, 0, 1, 2)
    return y
